# baseline (speedup 1.0000x reference)
import jax
import jax.numpy as jnp
from jax import lax
from jax.experimental import pallas as pl
from jax.experimental.pallas import tpu as pltpu
from jax.experimental.pallas import tpu_sc as plsc

T = 2048
H = 1024
E = 64
FF = 704
CAP = 40
THRESH = 0.2
DUMP = E * CAP
EB = E + 1
TBLK = 128
TPW = 64
CHK = 32


def _router_kernel(x_ref, wr_ref, slot_ref, xmul_ref, ymul_ref,
                   counts_ref, vcounts_ref):
    i = pl.program_id(0)
    x = x_ref[...]
    logits = jnp.dot(x, wr_ref[...], preferred_element_type=jnp.float32)
    m = jnp.max(logits, axis=1, keepdims=True)
    s = jnp.sum(jnp.exp(logits - m), axis=1, keepdims=True)
    topval = 1.0 / s
    e_iota = lax.broadcasted_iota(jnp.int32, (TBLK, E), 1)
    idx = jnp.min(jnp.where(logits == m, e_iota, E), axis=1, keepdims=True)
    oh = (e_iota == idx).astype(jnp.float32)

    @pl.when(i == 0)
    def _():
        counts_ref[...] = jnp.zeros((1, 1, E), jnp.int32)
        vcounts_ref[...] = jnp.zeros((1, 1, E), jnp.int32)

    carry = counts_ref[...].reshape(1, E).astype(jnp.float32)
    r_io = lax.broadcasted_iota(jnp.int32, (TBLK, TBLK), 0)
    c_io = lax.broadcasted_iota(jnp.int32, (TBLK, TBLK), 1)
    lstrict = (r_io > c_io).astype(jnp.float32)
    pref = jnp.dot(lstrict, oh, preferred_element_type=jnp.float32)
    pos = jnp.sum((pref + carry) * oh, axis=1, keepdims=True).astype(jnp.int32)
    counts_ref[...] = (carry + jnp.sum(oh, axis=0, keepdims=True)).astype(
        jnp.int32).reshape(1, 1, E)
    skip = topval < THRESH
    valid = jnp.logical_and(pos < CAP, jnp.logical_not(skip))
    slot_ref[...] = jnp.where(valid, idx * CAP + pos, DUMP)
    xmul_ref[...] = jnp.where(skip, 1.0, 0.0)
    ymul_ref[...] = jnp.where(valid, topval, 0.0)
    voh = oh * valid.astype(jnp.float32)
    vcounts_ref[...] = (vcounts_ref[...].reshape(1, E).astype(jnp.float32)
                        + jnp.sum(voh, axis=0, keepdims=True)).astype(
                            jnp.int32).reshape(1, 1, E)


def _mlp_kernel(sched_ref, xb_ref, wg_ref, wu_ref, wd_ref, ye_ref):
    i = pl.program_id(0)
    flag = sched_ref[EB + i]

    @pl.when(flag != 0)
    def _():
        xb = xb_ref[...]
        g = jnp.dot(xb, wg_ref[0], preferred_element_type=jnp.float32)
        u = jnp.dot(xb, wu_ref[0], preferred_element_type=jnp.float32)
        h = g * jax.nn.sigmoid(g) * u
        ye_ref[...] = jnp.dot(h, wd_ref[0], preferred_element_type=jnp.float32)

    @pl.when(flag == 0)
    def _():
        ye_ref[...] = jnp.zeros((CAP, H), jnp.float32)


def _sc_scatter(x_hbm, slot_hbm, buf_hbm, idx_v, x_v, sem):
    nc = 2
    wid = lax.axis_index("s") * nc + lax.axis_index("c")
    base = wid * TPW
    pltpu.sync_copy(slot_hbm.at[pl.ds(base, TPW)], idx_v)
    pltpu.sync_copy(x_hbm.at[pl.ds(base, TPW)], x_v)
    pltpu.async_copy(x_v, buf_hbm.at[idx_v], sem).wait()


def _dyn_pick(vec16, lanevec):
    dn = lax.GatherDimensionNumbers(
        offset_dims=(), collapsed_slice_dims=(0,), start_index_map=(0,))
    return lax.gather(vec16, lanevec[:, None], dn, (1,),
                      mode=lax.GatherScatterMode.PROMISE_IN_BOUNDS)


def _sc_gather(x_hbm, slot_hbm, xmul_hbm, ymul_hbm, ye_hbm,
               out_hbm, idx_v, xm_v, gm_v, x_v, y_v, sem):
    nc = 2
    wid = lax.axis_index("s") * nc + lax.axis_index("c")

    def chunk_body(c, carry):
        base = wid * TPW + c * CHK
        pltpu.sync_copy(slot_hbm.at[pl.ds(base, CHK)], idx_v)
        pltpu.sync_copy(xmul_hbm.at[pl.ds(base, CHK)], xm_v)
        pltpu.sync_copy(ymul_hbm.at[pl.ds(base, CHK)], gm_v)
        pltpu.sync_copy(x_hbm.at[pl.ds(base, CHK)], x_v)
        pltpu.async_copy(ye_hbm.at[idx_v], y_v, sem).wait()

        def half_body(h2, carry2):
            xm16 = xm_v[pl.ds(h2 * 16, 16)]
            gm16 = gm_v[pl.ds(h2 * 16, 16)]

            def row_body(lane, carry3):
                lanevec = jnp.full((16,), lane, dtype=jnp.int32)
                xm = _dyn_pick(xm16, lanevec)
                gm = _dyn_pick(gm16, lanevec)
                r = h2 * 16 + lane

                def grp_body(j, carry4):
                    xs = x_v[r, pl.ds(j * 16, 16)]
                    ys = y_v[r, pl.ds(j * 16, 16)]
                    x_v[r, pl.ds(j * 16, 16)] = xm * xs + gm * ys
                    return carry4

                return lax.fori_loop(0, H // 16, grp_body, carry3)

            return lax.fori_loop(0, 16, row_body, carry2)

        lax.fori_loop(0, CHK // 16, half_body, 0)
        pltpu.sync_copy(x_v, out_hbm.at[pl.ds(base, CHK)])
        return carry

    lax.fori_loop(0, TPW // CHK, chunk_body, 0)


def kernel(hidden_states, Wr, Wg, Wu, Wd):
    x2 = hidden_states.reshape(T, H)

    slot2, xmul2, ymul2, _counts3, vcounts3 = pl.pallas_call(
        _router_kernel,
        grid=(T // TBLK,),
        in_specs=[
            pl.BlockSpec((TBLK, H), lambda i: (i, 0)),
            pl.BlockSpec((H, E), lambda i: (0, 0)),
        ],
        out_specs=[
            pl.BlockSpec((TBLK, 1), lambda i: (i, 0)),
            pl.BlockSpec((TBLK, 1), lambda i: (i, 0)),
            pl.BlockSpec((TBLK, 1), lambda i: (i, 0)),
            pl.BlockSpec((1, 1, E), lambda i: (0, 0, 0)),
            pl.BlockSpec((1, 1, E), lambda i: (0, 0, 0)),
        ],
        out_shape=[
            jax.ShapeDtypeStruct((T, 1), jnp.int32),
            jax.ShapeDtypeStruct((T, 1), jnp.float32),
            jax.ShapeDtypeStruct((T, 1), jnp.float32),
            jax.ShapeDtypeStruct((1, 1, E), jnp.int32),
            jax.ShapeDtypeStruct((1, 1, E), jnp.int32),
        ],
    )(x2, Wr)
    slot1 = slot2.reshape(T)

    active = vcounts3.reshape(E) > 0
    iota = jnp.arange(E, dtype=jnp.int32)
    last_active = jnp.maximum(jnp.max(jnp.where(active, iota, -1)), 0)
    cand = jnp.where(active, iota, E)
    suffix_first = lax.cummin(cand[::-1])[::-1]
    windex = jnp.where(suffix_first < E, suffix_first, last_active)
    windex = jnp.concatenate([windex, last_active[None]]).astype(jnp.int32)
    aflag = jnp.concatenate(
        [active.astype(jnp.int32), jnp.zeros((1,), jnp.int32)])
    sched = jnp.concatenate([windex, aflag])

    buf = pl.kernel(
        _sc_scatter,
        out_type=jax.ShapeDtypeStruct((DUMP + 1, H), jnp.float32),
        mesh=plsc.VectorSubcoreMesh(core_axis_name="c", subcore_axis_name="s"),
        scratch_types=[
            pltpu.VMEM((TPW,), jnp.int32),
            pltpu.VMEM((TPW, H), jnp.float32),
            pltpu.SemaphoreType.DMA,
        ],
    )(x2, slot1)

    ye = pl.pallas_call(
        _mlp_kernel,
        grid_spec=pltpu.PrefetchScalarGridSpec(
            num_scalar_prefetch=1,
            grid=(EB,),
            in_specs=[
                pl.BlockSpec((CAP, H), lambda i, s: (s[i], 0)),
                pl.BlockSpec((1, H, FF), lambda i, s: (s[i], 0, 0)),
                pl.BlockSpec((1, H, FF), lambda i, s: (s[i], 0, 0)),
                pl.BlockSpec((1, FF, H), lambda i, s: (s[i], 0, 0)),
            ],
            out_specs=pl.BlockSpec((CAP, H), lambda i, s: (i, 0)),
        ),
        out_shape=jax.ShapeDtypeStruct((EB * CAP, H), jnp.float32),
    )(sched, buf, Wg, Wu, Wd)

    out = pl.kernel(
        _sc_gather,
        out_type=jax.ShapeDtypeStruct((T, H), jnp.float32),
        mesh=plsc.VectorSubcoreMesh(core_axis_name="c", subcore_axis_name="s"),
        scratch_types=[
            pltpu.VMEM((CHK,), jnp.int32),
            pltpu.VMEM((CHK,), jnp.float32),
            pltpu.VMEM((CHK,), jnp.float32),
            pltpu.VMEM((CHK, H), jnp.float32),
            pltpu.VMEM((CHK, H), jnp.float32),
            pltpu.SemaphoreType.DMA,
        ],
    )(x2, slot1, xmul2.reshape(T), ymul2.reshape(T), ye)

    return out.reshape(hidden_states.shape)
